# baseline (speedup 1.0000x reference)
import math

import jax
import jax.numpy as jnp
import numpy as np
from jax.experimental import pallas as pl

_FMAP_DIMS = [64, 32, 16, 8, 4]
_SCALES = [0.1, 0.2, 0.375, 0.55, 0.725]
_NCLS = 20
_FALPHA = 0.25
_EPS = 1e-9
_SIZES = [f * f for f in _FMAP_DIMS]
_OFFS = [0, 4096, 5120, 5376, 5440]
_TOTAL = 5456
_NOBJ = 8
_HALF = [float(np.float32(s) / np.float32(2.0)) for s in _SCALES]
_INVF = [float(np.float32(1.0) / np.float32(f)) for f in _FMAP_DIMS]


def _make_priors_np():
    ps = []
    for f, s in zip(_FMAP_DIMS, _SCALES):
        jj, ii = np.meshgrid(np.arange(f), np.arange(f), indexing="ij")
        cx = (ii.reshape(-1).astype(np.float32) + 0.5) / f
        cy = (jj.reshape(-1).astype(np.float32) + 0.5) / f
        p = np.stack(
            [cx, cy, np.full(f * f, s, np.float32), np.full(f * f, s, np.float32)],
            axis=1,
        )
        ps.append(np.clip(p, 0.0, 1.0))
    return np.concatenate(ps, axis=0)


_PRIORS = _make_priors_np()


def _atan(x):
    ax = jnp.abs(x)
    big = ax > 2.414213562373095
    mid = (ax > 0.4142135623730950) & (~big)
    xr = jnp.where(big, -1.0 / jnp.where(big, ax, 1.0),
                   jnp.where(mid, (ax - 1.0) / (ax + 1.0), ax))
    y0 = jnp.where(big, math.pi / 2.0, jnp.where(mid, math.pi / 4.0, 0.0))
    z = xr * xr
    p = (((8.05374449538e-2 * z - 1.38776856032e-1) * z + 1.99777106478e-1) * z
         - 3.33329491539e-1) * z * xr + xr
    r = y0 + p
    return jnp.where(x < 0.0, -r, r)


_IPB = 4


def _image(locs, st, bb, labs, prir):
    f32 = jnp.float32

    bx0 = bb[:, 0:1]; by0 = bb[:, 1:2]; bx1 = bb[:, 2:3]; by1 = bb[:, 3:4]
    bcx = (bx0 + bx1) / 2.0
    bcy = (by0 + by1) / 2.0
    aa = (bx1 - bx0) * (by1 - by0)

    lv_rank, lv_ov, lv_inside, lv_glob, lv_nw = [], [], [], [], []
    s1 = jnp.zeros((_NOBJ, 1), f32)
    for lv in range(5):
        f = _FMAP_DIMS[lv]
        half, invf = _HALF[lv], _INVF[lv]
        w = min(5, f)
        nw = w * w
        lane = jax.lax.broadcasted_iota(jnp.int32, (_NOBJ, nw), 1)
        dr = lane // w
        dc = lane % w
        cellx = jnp.clip((bcx * float(f)).astype(jnp.int32), 0, f - 1)
        celly = jnp.clip((bcy * float(f)).astype(jnp.int32), 0, f - 1)
        c0 = jnp.clip(cellx - 2, 0, f - w)
        r0 = jnp.clip(celly - 2, 0, f - w)
        rr = r0 + dr
        cc = c0 + dc
        cxw = (cc.astype(f32) + 0.5) * invf
        cyw = (rr.astype(f32) + 0.5) * invf
        dl = jnp.sqrt((bcx - cxw) ** 2 + (bcy - cyw) ** 2)
        rank = jnp.zeros((_NOBJ, nw), jnp.int32)
        for j in range(nw):
            dj = dl[:, j:j + 1]
            beats = (dj < dl) | ((dj == dl) & (lane > j))
            rank = rank + beats.astype(jnp.int32)
        is_cand = rank < 9
        px0 = cxw - half; px1 = cxw + half
        py0 = cyw - half; py1 = cyw + half
        iw = jnp.clip(jnp.minimum(bx1, px1) - jnp.maximum(bx0, px0), 0.0, None)
        ih = jnp.clip(jnp.minimum(by1, py1) - jnp.maximum(by0, py0), 0.0, None)
        inter = iw * ih
        ab = (px1 - px0) * (py1 - py0)
        ovc = inter / (aa + ab - inter + 1e-12)
        inside = (bx0 < cxw) & (cxw < bx1) & (by0 < cyw) & (cyw < by1)
        s1 = s1 + jnp.sum(jnp.where(is_cand, ovc, 0.0), axis=1, keepdims=True)
        lv_rank.append(rank); lv_ov.append(ovc); lv_inside.append(inside)
        lv_glob.append((rr * f + cc).astype(f32)); lv_nw.append(nw)

    mu = s1 / 45.0
    s2 = jnp.zeros((_NOBJ, 1), f32)
    for lv in range(5):
        s2 = s2 + jnp.sum(
            jnp.where(lv_rank[lv] < 9, (lv_ov[lv] - mu) ** 2, 0.0),
            axis=1, keepdims=True)
    thr = mu + jnp.sqrt(s2 / 44.0)

    cols_val, cols_idx = [], []
    for lv in range(5):
        rank, ovc, inside = lv_rank[lv], lv_ov[lv], lv_inside[lv]
        val_cell = jnp.where((ovc > thr) & inside, ovc, 0.0)
        for c in range(9):
            selc = rank == c
            cols_val.append(
                jnp.sum(jnp.where(selc, val_cell, 0.0), axis=1, keepdims=True))
            cols_idx.append(
                jnp.sum(jnp.where(selc, lv_glob[lv], 0.0), axis=1, keepdims=True)
                + _OFFS[lv])
    val_all = jnp.concatenate(cols_val, axis=1)
    cand_idx = jnp.concatenate(cols_idx, axis=1)

    val_t = jnp.transpose(val_all)
    idx_t = jnp.transpose(cand_idx)
    lane8 = jax.lax.broadcasted_iota(jnp.int32, (45, _NOBJ), 1)
    vmax_c = jnp.max(val_t, axis=1, keepdims=True)
    found_c = vmax_c > 0.0
    ob_c = jnp.min(jnp.where(val_t == vmax_c, lane8, _NOBJ), axis=1, keepdims=True)
    selobj = lane8 == ob_c
    pg_c = jnp.sum(jnp.where(selobj, idx_t, 0.0), axis=1, keepdims=True)
    lbl_c = jnp.sum(jnp.where(selobj, labs.astype(f32), 0.0), axis=1, keepdims=True)
    mf = jnp.where(found_c, 1.0, 0.0)

    lane_t = jax.lax.broadcasted_iota(jnp.int32, (45, _TOTAL), 1)
    pgi = pg_c.astype(jnp.int32)
    Pm = lane_t == pgi
    P = jnp.where(Pm, 1.0, 0.0)
    M45 = Pm & found_c
    step_c = jax.lax.broadcasted_iota(jnp.int32, (45, 1), 0)
    lbl_i = lbl_c.astype(jnp.int32)
    e_c = step_c * 64 + lbl_i
    tcenc = jnp.max(jnp.where(M45, e_c, -1), axis=0, keepdims=True)
    npos = jnp.sum(jnp.where(tcenc >= 0, 1.0, 0.0))
    is_writer = (jnp.sum(jnp.where(Pm & (tcenc == e_c), 1.0, 0.0),
                         axis=1, keepdims=True) > 0.0) & found_c

    g = jnp.dot(P, locs, preferred_element_type=f32)
    pri = jnp.dot(P, prir, preferred_element_type=f32)
    tbx = jnp.dot(jnp.where(selobj, 1.0, 0.0), bb,
                  preferred_element_type=f32)

    dcx = g[:, 0:1] * pri[:, 2:3] / 10.0 + pri[:, 0:1]
    dcy = g[:, 1:2] * pri[:, 3:4] / 10.0 + pri[:, 1:2]
    dw = jnp.exp(g[:, 2:3] / 5.0) * pri[:, 2:3]
    dh = jnp.exp(g[:, 3:4] / 5.0) * pri[:, 3:4]
    dx0 = dcx - dw / 2.0; dy0 = dcy - dh / 2.0
    dx1 = dcx + dw / 2.0; dy1 = dcy + dh / 2.0
    tx0 = tbx[:, 0:1]; ty0 = tbx[:, 1:2]; tx1 = tbx[:, 2:3]; ty1 = tbx[:, 3:4]

    iw2 = jnp.clip(jnp.minimum(dx1, tx1) - jnp.maximum(dx0, tx0), 0.0, None)
    ih2 = jnp.clip(jnp.minimum(dy1, ty1) - jnp.maximum(dy0, ty0), 0.0, None)
    inter2 = iw2 * ih2
    ap = (dx1 - dx0) * (dy1 - dy0)
    at = (tx1 - tx0) * (ty1 - ty0)
    iou = inter2 / (ap + at - inter2 + _EPS)
    cw = jnp.maximum(dx1, tx1) - jnp.minimum(dx0, tx0)
    ch = jnp.maximum(dy1, ty1) - jnp.minimum(dy0, ty0)
    c2 = cw ** 2 + ch ** 2 + _EPS
    rho2 = (((dx0 + dx1) - (tx0 + tx1)) / 2.0) ** 2 + (((dy0 + dy1) - (ty0 + ty1)) / 2.0) ** 2
    wp = dx1 - dx0; hp = dy1 - dy0
    wt = tx1 - tx0; ht = ty1 - ty0
    v_ar = (4.0 / (math.pi ** 2)) * (
        _atan(wt / (ht + _EPS)) - _atan(wp / (hp + _EPS))) ** 2
    alpha = v_ar / (1.0 - iou + v_ar + _EPS)
    ciou = iou - rho2 / c2 - alpha * v_ar
    ciou_sum = jnp.sum((1.0 - ciou) * mf)
    mask_sum = jnp.sum(mf)

    pp = 1.0 / (1.0 + jnp.exp(-st))
    t2 = pp ** 2 * jnp.log(jnp.clip(1.0 - pp, 1e-12, 1.0))
    main = -(1.0 - _FALPHA) * jnp.sum(t2)
    c45 = jax.lax.dot_general(P, st, (((1,), (0,)), ((), ())),
                              preferred_element_type=f32)
    cls20 = jax.lax.broadcasted_iota(jnp.int32, (45, _NCLS), 1)
    xsel = jnp.sum(jnp.where(cls20 == lbl_i - 1, c45, 0.0), axis=1, keepdims=True)
    pp2 = 1.0 / (1.0 + jnp.exp(-xsel))
    t1s = (1.0 - pp2) ** 2 * jnp.log(jnp.clip(pp2, 1e-12, 1.0))
    t2s = pp2 ** 2 * jnp.log(jnp.clip(1.0 - pp2, 1e-12, 1.0))
    corr = jnp.sum(jnp.where(is_writer,
                             (1.0 - _FALPHA) * t2s - _FALPHA * t1s, 0.0))
    fsum = main + corr
    return fsum, npos, ciou_sum, mask_sum


def _body(locs_ref, scores_ref, boxes_ref, labels_ref, prir_ref, out_ref):
    prir = prir_ref[...]
    fsum = 0.0
    npos = 0.0
    csum = 0.0
    msum = 0.0
    for img in range(_IPB):
        fs, np_, cs, ms = _image(locs_ref[img], scores_ref[img],
                                 boxes_ref[img], labels_ref[img], prir)
        fsum += fs; npos += np_; csum += cs; msum += ms

    lane128 = jax.lax.broadcasted_iota(jnp.int32, (1, 128), 1)
    vec = (jnp.where(lane128 == 0, fsum, 0.0)
           + jnp.where(lane128 == 1, npos, 0.0)
           + jnp.where(lane128 == 2, csum, 0.0)
           + jnp.where(lane128 == 3, msum, 0.0))
    b = pl.program_id(0)

    @pl.when(b == 0)
    def _init():
        out_ref[0] = vec

    @pl.when(b != 0)
    def _acc():
        out_ref[0] = out_ref[0] + vec


def kernel(predicted_locs, predicted_scores, boxes, labels):
    B = predicted_locs.shape[0]
    labels3 = labels.astype(jnp.int32).reshape(B, 1, _NOBJ)
    pri_r = jnp.asarray(_PRIORS)
    out = pl.pallas_call(
        _body,
        grid=(B // _IPB,),
        in_specs=[
            pl.BlockSpec((_IPB, _TOTAL, 4), lambda b: (b, 0, 0)),
            pl.BlockSpec((_IPB, _TOTAL, _NCLS), lambda b: (b, 0, 0)),
            pl.BlockSpec((_IPB, _NOBJ, 4), lambda b: (b, 0, 0)),
            pl.BlockSpec((_IPB, 1, _NOBJ), lambda b: (b, 0, 0)),
            pl.BlockSpec((_TOTAL, 4), lambda b: (0, 0)),
        ],
        out_specs=pl.BlockSpec((1, 1, 128), lambda b: (0, 0, 0)),
        out_shape=jax.ShapeDtypeStruct((1, 1, 128), jnp.float32),
    )(predicted_locs, predicted_scores, boxes, labels3, pri_r)
    r = out[0, 0]
    return r[0] / r[1] + r[2] / r[3]


# speedup vs baseline: 1.4389x; 1.4389x over previous
import math

import jax
import jax.numpy as jnp
import numpy as np
from jax.experimental import pallas as pl

_FMAP_DIMS = [64, 32, 16, 8, 4]
_SCALES = [0.1, 0.2, 0.375, 0.55, 0.725]
_NCLS = 20
_FALPHA = 0.25
_EPS = 1e-9
_SIZES = [f * f for f in _FMAP_DIMS]
_OFFS = [0, 4096, 5120, 5376, 5440]
_TOTAL = 5456
_NOBJ = 8
_HALF = [float(np.float32(s) / np.float32(2.0)) for s in _SCALES]
_INVF = [float(np.float32(1.0) / np.float32(f)) for f in _FMAP_DIMS]


def _make_priors_np():
    ps = []
    for f, s in zip(_FMAP_DIMS, _SCALES):
        jj, ii = np.meshgrid(np.arange(f), np.arange(f), indexing="ij")
        cx = (ii.reshape(-1).astype(np.float32) + 0.5) / f
        cy = (jj.reshape(-1).astype(np.float32) + 0.5) / f
        p = np.stack(
            [cx, cy, np.full(f * f, s, np.float32), np.full(f * f, s, np.float32)],
            axis=1,
        )
        ps.append(np.clip(p, 0.0, 1.0))
    return np.concatenate(ps, axis=0)


_PRIORS = _make_priors_np()


def _atan(x):
    ax = jnp.abs(x)
    big = ax > 2.414213562373095
    mid = (ax > 0.4142135623730950) & (~big)
    xr = jnp.where(big, -1.0 / jnp.where(big, ax, 1.0),
                   jnp.where(mid, (ax - 1.0) / (ax + 1.0), ax))
    y0 = jnp.where(big, math.pi / 2.0, jnp.where(mid, math.pi / 4.0, 0.0))
    z = xr * xr
    p = (((8.05374449538e-2 * z - 1.38776856032e-1) * z + 1.99777106478e-1) * z
         - 3.33329491539e-1) * z * xr + xr
    r = y0 + p
    return jnp.where(x < 0.0, -r, r)


_IPB = 4


def _image(locs, st, bb, labs, prir):
    f32 = jnp.float32

    bx0 = bb[:, 0:1]; by0 = bb[:, 1:2]; bx1 = bb[:, 2:3]; by1 = bb[:, 3:4]
    bcx = (bx0 + bx1) / 2.0
    bcy = (by0 + by1) / 2.0
    aa = (bx1 - bx0) * (by1 - by0)

    lv_rank, lv_ov, lv_inside, lv_glob, lv_nw = [], [], [], [], []
    s1 = jnp.zeros((_NOBJ, 1), f32)
    for lv in range(5):
        f = _FMAP_DIMS[lv]
        half, invf = _HALF[lv], _INVF[lv]
        w = min(5, f)
        nw = w * w
        lane = jax.lax.broadcasted_iota(jnp.int32, (_NOBJ, nw), 1)
        dr = lane // w
        dc = lane % w
        cellx = jnp.clip((bcx * float(f)).astype(jnp.int32), 0, f - 1)
        celly = jnp.clip((bcy * float(f)).astype(jnp.int32), 0, f - 1)
        c0 = jnp.clip(cellx - 2, 0, f - w)
        r0 = jnp.clip(celly - 2, 0, f - w)
        rr = r0 + dr
        cc = c0 + dc
        cxw = (cc.astype(f32) + 0.5) * invf
        cyw = (rr.astype(f32) + 0.5) * invf
        dl = jnp.sqrt((bcx - cxw) ** 2 + (bcy - cyw) ** 2)
        rank = jnp.zeros((_NOBJ, nw), jnp.int32)
        for j in range(nw):
            dj = dl[:, j:j + 1]
            beats = (dj < dl) | ((dj == dl) & (lane > j))
            rank = rank + beats.astype(jnp.int32)
        is_cand = rank < 9
        px0 = cxw - half; px1 = cxw + half
        py0 = cyw - half; py1 = cyw + half
        iw = jnp.clip(jnp.minimum(bx1, px1) - jnp.maximum(bx0, px0), 0.0, None)
        ih = jnp.clip(jnp.minimum(by1, py1) - jnp.maximum(by0, py0), 0.0, None)
        inter = iw * ih
        ab = (px1 - px0) * (py1 - py0)
        ovc = inter / (aa + ab - inter + 1e-12)
        inside = (bx0 < cxw) & (cxw < bx1) & (by0 < cyw) & (cyw < by1)
        s1 = s1 + jnp.sum(jnp.where(is_cand, ovc, 0.0), axis=1, keepdims=True)
        lv_rank.append(rank); lv_ov.append(ovc); lv_inside.append(inside)
        lv_glob.append((rr * f + cc).astype(f32)); lv_nw.append(nw)

    mu = s1 / 45.0
    s2 = jnp.zeros((_NOBJ, 1), f32)
    for lv in range(5):
        s2 = s2 + jnp.sum(
            jnp.where(lv_rank[lv] < 9, (lv_ov[lv] - mu) ** 2, 0.0),
            axis=1, keepdims=True)
    thr = mu + jnp.sqrt(s2 / 44.0)

    cols_val, cols_idx = [], []
    for lv in range(5):
        rank, ovc, inside = lv_rank[lv], lv_ov[lv], lv_inside[lv]
        val_cell = jnp.where((ovc > thr) & inside, ovc, 0.0)
        for c in range(9):
            selc = rank == c
            cols_val.append(
                jnp.sum(jnp.where(selc, val_cell, 0.0), axis=1, keepdims=True))
            cols_idx.append(
                jnp.sum(jnp.where(selc, lv_glob[lv], 0.0), axis=1, keepdims=True)
                + _OFFS[lv])
    val_all = jnp.concatenate(cols_val, axis=1)
    cand_idx = jnp.concatenate(cols_idx, axis=1)

    val_t = jnp.transpose(val_all)
    idx_t = jnp.transpose(cand_idx)
    lane8 = jax.lax.broadcasted_iota(jnp.int32, (45, _NOBJ), 1)
    vmax_c = jnp.max(val_t, axis=1, keepdims=True)
    found_c = vmax_c > 0.0
    ob_c = jnp.min(jnp.where(val_t == vmax_c, lane8, _NOBJ), axis=1, keepdims=True)
    selobj = lane8 == ob_c
    pg_c = jnp.sum(jnp.where(selobj, idx_t, 0.0), axis=1, keepdims=True)
    lbl_c = jnp.sum(jnp.where(selobj, labs.astype(f32), 0.0), axis=1, keepdims=True)
    mf = jnp.where(found_c, 1.0, 0.0)

    lane_t = jax.lax.broadcasted_iota(jnp.int32, (45, _TOTAL), 1)
    pgi = pg_c.astype(jnp.int32)
    P = jnp.where(lane_t == pgi, 1.0, 0.0)
    step_c = jax.lax.broadcasted_iota(jnp.int32, (45, 1), 0)
    step_r = jax.lax.broadcasted_iota(jnp.int32, (45, 45), 1)
    pg_row = jnp.transpose(pg_c)
    mf_row = jnp.transpose(mf)
    over = (pg_row == pg_c) & (step_r > step_c) & (mf_row > 0.0)
    dup_later = jnp.sum(jnp.where(over, 1.0, 0.0), axis=1, keepdims=True) > 0.0
    is_writer = found_c & jnp.logical_not(dup_later)
    lbl_i = lbl_c.astype(jnp.int32)
    npos = jnp.sum(jnp.where(is_writer, 1.0, 0.0))

    g = jnp.dot(P, locs, preferred_element_type=f32)
    pri = jnp.dot(P, prir, preferred_element_type=f32)
    tbx = jnp.dot(jnp.where(selobj, 1.0, 0.0), bb,
                  preferred_element_type=f32)

    dcx = g[:, 0:1] * pri[:, 2:3] / 10.0 + pri[:, 0:1]
    dcy = g[:, 1:2] * pri[:, 3:4] / 10.0 + pri[:, 1:2]
    dw = jnp.exp(g[:, 2:3] / 5.0) * pri[:, 2:3]
    dh = jnp.exp(g[:, 3:4] / 5.0) * pri[:, 3:4]
    dx0 = dcx - dw / 2.0; dy0 = dcy - dh / 2.0
    dx1 = dcx + dw / 2.0; dy1 = dcy + dh / 2.0
    tx0 = tbx[:, 0:1]; ty0 = tbx[:, 1:2]; tx1 = tbx[:, 2:3]; ty1 = tbx[:, 3:4]

    iw2 = jnp.clip(jnp.minimum(dx1, tx1) - jnp.maximum(dx0, tx0), 0.0, None)
    ih2 = jnp.clip(jnp.minimum(dy1, ty1) - jnp.maximum(dy0, ty0), 0.0, None)
    inter2 = iw2 * ih2
    ap = (dx1 - dx0) * (dy1 - dy0)
    at = (tx1 - tx0) * (ty1 - ty0)
    iou = inter2 / (ap + at - inter2 + _EPS)
    cw = jnp.maximum(dx1, tx1) - jnp.minimum(dx0, tx0)
    ch = jnp.maximum(dy1, ty1) - jnp.minimum(dy0, ty0)
    c2 = cw ** 2 + ch ** 2 + _EPS
    rho2 = (((dx0 + dx1) - (tx0 + tx1)) / 2.0) ** 2 + (((dy0 + dy1) - (ty0 + ty1)) / 2.0) ** 2
    wp = dx1 - dx0; hp = dy1 - dy0
    wt = tx1 - tx0; ht = ty1 - ty0
    v_ar = (4.0 / (math.pi ** 2)) * (
        _atan(wt / (ht + _EPS)) - _atan(wp / (hp + _EPS))) ** 2
    alpha = v_ar / (1.0 - iou + v_ar + _EPS)
    ciou = iou - rho2 / c2 - alpha * v_ar
    ciou_sum = jnp.sum((1.0 - ciou) * mf)
    mask_sum = jnp.sum(mf)

    pp = 1.0 / (1.0 + jnp.exp(-st))
    t2 = pp ** 2 * jnp.log(jnp.clip(1.0 - pp, 1e-12, 1.0))
    main = -(1.0 - _FALPHA) * jnp.sum(t2)
    c45 = jax.lax.dot_general(P, st, (((1,), (1,)), ((), ())),
                              preferred_element_type=f32)
    cls20 = jax.lax.broadcasted_iota(jnp.int32, (45, _NCLS), 1)
    xsel = jnp.sum(jnp.where(cls20 == lbl_i - 1, c45, 0.0), axis=1, keepdims=True)
    pp2 = 1.0 / (1.0 + jnp.exp(-xsel))
    t1s = (1.0 - pp2) ** 2 * jnp.log(jnp.clip(pp2, 1e-12, 1.0))
    t2s = pp2 ** 2 * jnp.log(jnp.clip(1.0 - pp2, 1e-12, 1.0))
    corr = jnp.sum(jnp.where(is_writer,
                             (1.0 - _FALPHA) * t2s - _FALPHA * t1s, 0.0))
    fsum = main + corr
    return fsum, npos, ciou_sum, mask_sum


def _body(locs_ref, scores_ref, boxes_ref, labels_ref, prir_ref, out_ref):
    prir = prir_ref[...]
    fsum = 0.0
    npos = 0.0
    csum = 0.0
    msum = 0.0
    for img in range(_IPB):
        fs, np_, cs, ms = _image(locs_ref[img], scores_ref[img],
                                 boxes_ref[img], labels_ref[img], prir)
        fsum += fs; npos += np_; csum += cs; msum += ms

    lane128 = jax.lax.broadcasted_iota(jnp.int32, (1, 128), 1)
    vec = (jnp.where(lane128 == 0, fsum, 0.0)
           + jnp.where(lane128 == 1, npos, 0.0)
           + jnp.where(lane128 == 2, csum, 0.0)
           + jnp.where(lane128 == 3, msum, 0.0))
    b = pl.program_id(0)

    @pl.when(b == 0)
    def _init():
        out_ref[0] = vec

    @pl.when(b != 0)
    def _acc():
        out_ref[0] = out_ref[0] + vec


def kernel(predicted_locs, predicted_scores, boxes, labels):
    B = predicted_locs.shape[0]
    scores_t = jnp.transpose(predicted_scores, (0, 2, 1))
    labels3 = labels.astype(jnp.int32).reshape(B, 1, _NOBJ)
    pri_r = jnp.asarray(_PRIORS)
    out = pl.pallas_call(
        _body,
        grid=(B // _IPB,),
        in_specs=[
            pl.BlockSpec((_IPB, _TOTAL, 4), lambda b: (b, 0, 0)),
            pl.BlockSpec((_IPB, _NCLS, _TOTAL), lambda b: (b, 0, 0)),
            pl.BlockSpec((_IPB, _NOBJ, 4), lambda b: (b, 0, 0)),
            pl.BlockSpec((_IPB, 1, _NOBJ), lambda b: (b, 0, 0)),
            pl.BlockSpec((_TOTAL, 4), lambda b: (0, 0)),
        ],
        out_specs=pl.BlockSpec((1, 1, 128), lambda b: (0, 0, 0)),
        out_shape=jax.ShapeDtypeStruct((1, 1, 128), jnp.float32),
    )(predicted_locs, scores_t, boxes, labels3, pri_r)
    r = out[0, 0]
    return r[0] / r[1] + r[2] / r[3]
